# Initial kernel scaffold; baseline (speedup 1.0000x reference)
import jax
import jax.numpy as jnp
from jax.experimental import pallas as pl


def kernel(Z, W):
    raise NotImplementedError("write your pallas kernel here")


# speedup vs baseline: 1.8107x; 1.8107x over previous
import functools

import jax
import jax.numpy as jnp
from jax import lax
from jax.experimental import pallas as pl
from jax.experimental.pallas import tpu as pltpu
from jax.experimental.pallas import tpu_sc as plsc

N_ATOMS = 100000
EMB = 128
CHUNK = 128

_info = plsc.get_sparse_core_info()
NC = _info.num_cores
NS = _info.num_subcores
NW = NC * NS

N_FULL_CHUNKS = N_ATOMS // CHUNK
REM = N_ATOMS - N_FULL_CHUNKS * CHUNK
REM_BASE = N_FULL_CHUNKS * CHUNK
REM_WORKER = N_FULL_CHUNKS % NW


def _make_lookup():
    mesh = plsc.VectorSubcoreMesh(core_axis_name="c", subcore_axis_name="s")

    @functools.partial(
        pl.kernel,
        mesh=mesh,
        out_type=jax.ShapeDtypeStruct((N_ATOMS, EMB), jnp.float32),
        scratch_types=[
            pltpu.VMEM((CHUNK,), jnp.int32),
            pltpu.VMEM((CHUNK, EMB), jnp.float32),
            pltpu.VMEM((REM,), jnp.int32),
            pltpu.VMEM((REM, EMB), jnp.float32),
            pltpu.SemaphoreType.DMA,
        ],
    )
    def lookup(z_hbm, table_hbm, out_hbm, idx_v, rows_v, idx_r, rows_r, sem):
        wid = lax.axis_index("s") * NC + lax.axis_index("c")
        n_chunks = jnp.where(wid < REM_WORKER, N_FULL_CHUNKS // NW + 1,
                             N_FULL_CHUNKS // NW)

        def body(k, carry):
            base = (wid + k * NW) * CHUNK
            pltpu.sync_copy(z_hbm.at[pl.ds(base, CHUNK)], idx_v)
            pltpu.async_copy(table_hbm.at[idx_v], rows_v, sem).wait()
            pltpu.sync_copy(rows_v, out_hbm.at[pl.ds(base, CHUNK)])
            return carry

        lax.fori_loop(0, n_chunks, body, 0)

        @pl.when(wid == REM_WORKER)
        def _():
            pltpu.sync_copy(z_hbm.at[pl.ds(REM_BASE, REM)], idx_r)
            pltpu.async_copy(table_hbm.at[idx_r], rows_r, sem).wait()
            pltpu.sync_copy(rows_r, out_hbm.at[pl.ds(REM_BASE, REM)])

    return lookup


_lookup = _make_lookup()


def kernel(Z, W):
    W_pad = jnp.concatenate([jnp.zeros((1, EMB), jnp.float32), W], axis=0)
    return _lookup(Z, W_pad)


# speedup vs baseline: 3.4997x; 1.9328x over previous
import functools

import jax
import jax.numpy as jnp
from jax import lax
from jax.experimental import pallas as pl
from jax.experimental.pallas import tpu as pltpu
from jax.experimental.pallas import tpu_sc as plsc

N_ATOMS = 100000
EMB = 128
TABLE_ROWS = 101
CHUNK = 128

_info = plsc.get_sparse_core_info()
NC = _info.num_cores
NS = _info.num_subcores
NW = NC * NS

N_FULL_CHUNKS = N_ATOMS // CHUNK
REM = N_ATOMS - N_FULL_CHUNKS * CHUNK
REM_BASE = N_FULL_CHUNKS * CHUNK
REM_WORKER = N_FULL_CHUNKS % NW


def _make_lookup():
    mesh = plsc.VectorSubcoreMesh(core_axis_name="c", subcore_axis_name="s")

    @functools.partial(
        pl.kernel,
        mesh=mesh,
        out_type=jax.ShapeDtypeStruct((N_ATOMS, EMB), jnp.float32),
        scratch_types=[
            pltpu.VMEM((CHUNK,), jnp.int32),
            pltpu.VMEM((CHUNK, EMB), jnp.float32),
            pltpu.VMEM((REM,), jnp.int32),
            pltpu.VMEM((REM, EMB), jnp.float32),
            pltpu.VMEM_SHARED((TABLE_ROWS, EMB), jnp.float32),
            pltpu.SemaphoreType.DMA,
        ],
    )
    def lookup(z_hbm, table_hbm, out_hbm, idx_v, rows_v, idx_r, rows_r,
               table_sh, sem):
        sid = lax.axis_index("s")
        wid = sid * NC + lax.axis_index("c")

        @pl.when(sid == 0)
        def _():
            pltpu.sync_copy(table_hbm, table_sh)

        plsc.subcore_barrier()

        n_chunks = jnp.where(wid < REM_WORKER, N_FULL_CHUNKS // NW + 1,
                             N_FULL_CHUNKS // NW)

        def body(k, carry):
            base = (wid + k * NW) * CHUNK
            pltpu.sync_copy(z_hbm.at[pl.ds(base, CHUNK)], idx_v)
            pltpu.async_copy(table_sh.at[idx_v], rows_v, sem).wait()
            pltpu.sync_copy(rows_v, out_hbm.at[pl.ds(base, CHUNK)])
            return carry

        lax.fori_loop(0, n_chunks, body, 0)

        @pl.when(wid == REM_WORKER)
        def _():
            pltpu.sync_copy(z_hbm.at[pl.ds(REM_BASE, REM)], idx_r)
            pltpu.async_copy(table_sh.at[idx_r], rows_r, sem).wait()
            pltpu.sync_copy(rows_r, out_hbm.at[pl.ds(REM_BASE, REM)])

    return lookup


_lookup = _make_lookup()


def kernel(Z, W):
    W_pad = jnp.concatenate([jnp.zeros((1, EMB), jnp.float32), W], axis=0)
    return _lookup(Z, W_pad)


# speedup vs baseline: 4.4215x; 1.2634x over previous
import functools

import jax
import jax.numpy as jnp
from jax import lax
from jax.experimental import pallas as pl
from jax.experimental.pallas import tpu as pltpu
from jax.experimental.pallas import tpu_sc as plsc

N_ATOMS = 100000
EMB = 128
TABLE_ROWS = 101
CHUNK = 128

_info = plsc.get_sparse_core_info()
NC = _info.num_cores
NS = _info.num_subcores
NW = NC * NS

N_UNITS = -(-N_ATOMS // CHUNK)
PAIRS = (N_UNITS // NW) // 2
EXTRA_W = N_UNITS % NW


def _make_lookup():
    mesh = plsc.VectorSubcoreMesh(core_axis_name="c", subcore_axis_name="s")

    @functools.partial(
        pl.kernel,
        mesh=mesh,
        out_type=jax.ShapeDtypeStruct((N_ATOMS, EMB), jnp.float32),
        scratch_types=[
            pltpu.VMEM((CHUNK,), jnp.int32),
            pltpu.VMEM((CHUNK,), jnp.int32),
            pltpu.VMEM((CHUNK, EMB), jnp.float32),
            pltpu.VMEM((CHUNK, EMB), jnp.float32),
            pltpu.VMEM_SHARED((TABLE_ROWS, EMB), jnp.float32),
            pltpu.SemaphoreType.DMA,
            pltpu.SemaphoreType.DMA,
            pltpu.SemaphoreType.DMA,
        ],
    )
    def lookup(z_hbm, table_hbm, out_hbm, idx0, idx1, rows0, rows1,
               table_sh, gsem, wsem0, wsem1):
        sid = lax.axis_index("s")
        wid = sid * NC + lax.axis_index("c")

        @pl.when(sid == 0)
        def _():
            pltpu.sync_copy(table_hbm, table_sh)

        plsc.subcore_barrier()

        idx = (idx0, idx1)
        rows = (rows0, rows1)
        wsem = (wsem0, wsem1)

        def unit_base(k):
            return jnp.minimum((wid + k * NW) * CHUNK, N_ATOMS - CHUNK)

        def pair_body(p, carry):
            for b in range(2):
                base = unit_base(2 * p + b)
                pltpu.sync_copy(z_hbm.at[pl.ds(base, CHUNK)], idx[b])

                @pl.when(p > 0)
                def _():
                    pltpu.make_async_copy(
                        out_hbm.at[pl.ds(0, CHUNK)], rows[b], wsem[b]).wait()

                pltpu.async_copy(table_sh.at[idx[b]], rows[b], gsem).wait()
                pltpu.async_copy(rows[b], out_hbm.at[pl.ds(base, CHUNK)],
                                 wsem[b])
            return carry

        lax.fori_loop(0, PAIRS, pair_body, 0)

        for b in range(2):
            pltpu.make_async_copy(
                out_hbm.at[pl.ds(0, CHUNK)], rows[b], wsem[b]).wait()

        @pl.when(wid < EXTRA_W)
        def _():
            base = unit_base(2 * PAIRS)
            pltpu.sync_copy(z_hbm.at[pl.ds(base, CHUNK)], idx0)
            pltpu.async_copy(table_sh.at[idx0], rows0, gsem).wait()
            pltpu.sync_copy(rows0, out_hbm.at[pl.ds(base, CHUNK)])

    return lookup


_lookup = _make_lookup()


def kernel(Z, W):
    W_pad = jnp.concatenate([jnp.zeros((1, EMB), jnp.float32), W], axis=0)
    return _lookup(Z, W_pad)


# speedup vs baseline: 5.4581x; 1.2345x over previous
import functools

import jax
import jax.numpy as jnp
from jax import lax
from jax.experimental import pallas as pl
from jax.experimental.pallas import tpu as pltpu
from jax.experimental.pallas import tpu_sc as plsc

N_ATOMS = 100000
EMB = 128
TABLE_ROWS = 101
CHUNK = 128

_info = plsc.get_sparse_core_info()
NC = _info.num_cores
NS = _info.num_subcores
NW = NC * NS

CHUNKS_PER_W = -(-N_ATOMS // (CHUNK * NW))
SPAN = CHUNKS_PER_W * CHUNK
PAIRS = CHUNKS_PER_W // 2


def _make_lookup():
    mesh = plsc.VectorSubcoreMesh(core_axis_name="c", subcore_axis_name="s")

    @functools.partial(
        pl.kernel,
        mesh=mesh,
        out_type=jax.ShapeDtypeStruct((N_ATOMS, EMB), jnp.float32),
        scratch_types=[
            pltpu.VMEM((SPAN,), jnp.int32),
            pltpu.VMEM((CHUNK, EMB), jnp.float32),
            pltpu.VMEM((CHUNK, EMB), jnp.float32),
            pltpu.VMEM_SHARED((TABLE_ROWS, EMB), jnp.float32),
            pltpu.SemaphoreType.DMA,
            pltpu.SemaphoreType.DMA,
            pltpu.SemaphoreType.DMA,
        ],
    )
    def lookup(z_hbm, table_hbm, out_hbm, idx_all, rows0, rows1,
               table_sh, gsem, wsem0, wsem1):
        sid = lax.axis_index("s")
        wid = sid * NC + lax.axis_index("c")

        @pl.when(sid == 0)
        def _():
            pltpu.sync_copy(table_hbm, table_sh)

        start = jnp.minimum(wid * SPAN, N_ATOMS - SPAN)
        pltpu.sync_copy(z_hbm.at[pl.ds(start, SPAN)], idx_all)

        plsc.subcore_barrier()

        rows = (rows0, rows1)
        wsem = (wsem0, wsem1)

        def gather(k, buf):
            pltpu.async_copy(
                table_sh.at[idx_all.at[pl.ds(k * CHUNK, CHUNK)]],
                rows[buf], gsem).wait()

        def pair_body(p, carry):
            for b in range(2):
                @pl.when(p > 0)
                def _():
                    pltpu.make_async_copy(
                        out_hbm.at[pl.ds(0, CHUNK)], rows[b], wsem[b]).wait()

                k = 2 * p + b
                gather(k, b)
                pltpu.async_copy(rows[b],
                                 out_hbm.at[pl.ds(start + k * CHUNK, CHUNK)],
                                 wsem[b])
            return carry

        lax.fori_loop(0, PAIRS, pair_body, 0)

        for b in range(2):
            pltpu.make_async_copy(
                out_hbm.at[pl.ds(0, CHUNK)], rows[b], wsem[b]).wait()

        k_last = 2 * PAIRS
        gather(k_last, 0)
        pltpu.sync_copy(rows0, out_hbm.at[pl.ds(start + k_last * CHUNK, CHUNK)])

    return lookup


_lookup = _make_lookup()


def kernel(Z, W):
    W_pad = jnp.concatenate([jnp.zeros((1, EMB), jnp.float32), W], axis=0)
    return _lookup(Z, W_pad)


# speedup vs baseline: 5.6646x; 1.0378x over previous
import functools

import jax
import jax.numpy as jnp
from jax import lax
from jax.experimental import pallas as pl
from jax.experimental.pallas import tpu as pltpu
from jax.experimental.pallas import tpu_sc as plsc

N_ATOMS = 100000
EMB = 128
TABLE_ROWS = 101
CHUNK = 128

_info = plsc.get_sparse_core_info()
NC = _info.num_cores
NS = _info.num_subcores
NW = NC * NS

CHUNKS_PER_W = -(-N_ATOMS // (CHUNK * NW))
SPAN = CHUNKS_PER_W * CHUNK
PAIRS = CHUNKS_PER_W // 2


def _make_lookup():
    mesh = plsc.VectorSubcoreMesh(core_axis_name="c", subcore_axis_name="s")

    @functools.partial(
        pl.kernel,
        mesh=mesh,
        out_type=jax.ShapeDtypeStruct((N_ATOMS, EMB), jnp.float32),
        scratch_types=[
            pltpu.VMEM((SPAN,), jnp.int32),
            pltpu.VMEM((CHUNK, EMB), jnp.float32),
            pltpu.VMEM((CHUNK, EMB), jnp.float32),
            pltpu.VMEM_SHARED((TABLE_ROWS, EMB), jnp.float32),
            pltpu.SemaphoreType.DMA,
            pltpu.SemaphoreType.DMA,
            pltpu.SemaphoreType.DMA,
            pltpu.SemaphoreType.DMA,
        ],
    )
    def lookup(z_hbm, table_hbm, out_hbm, idx_all, rows0, rows1,
               table_sh, gsem0, gsem1, wsem0, wsem1):
        sid = lax.axis_index("s")
        wid = sid * NC + lax.axis_index("c")

        @pl.when(sid == 0)
        def _():
            pltpu.sync_copy(table_hbm, table_sh)

        start = jnp.minimum(wid * SPAN, N_ATOMS - SPAN)
        pltpu.sync_copy(z_hbm.at[pl.ds(start, SPAN)], idx_all)

        plsc.subcore_barrier()

        rows = (rows0, rows1)
        gsem = (gsem0, gsem1)
        wsem = (wsem0, wsem1)

        def issue_gather(k, b):
            pltpu.async_copy(
                table_sh.at[idx_all.at[pl.ds(k * CHUNK, CHUNK)]],
                rows[b], gsem[b])

        def drain(sem, b):
            pltpu.make_async_copy(out_hbm.at[pl.ds(0, CHUNK)], rows[b],
                                  sem).wait()

        issue_gather(0, 0)

        def pair_body(p, carry):
            for b in range(2):
                k = 2 * p + b
                if b == 0:
                    @pl.when(p > 0)
                    def _():
                        drain(wsem[1], 1)
                else:
                    drain(wsem[0], 0)
                issue_gather(k + 1, 1 - b)
                drain(gsem[b], b)
                pltpu.async_copy(rows[b],
                                 out_hbm.at[pl.ds(start + k * CHUNK, CHUNK)],
                                 wsem[b])
            return carry

        lax.fori_loop(0, PAIRS, pair_body, 0)

        k_last = 2 * PAIRS
        drain(wsem[1], 1)
        drain(gsem[0], 0)
        pltpu.sync_copy(rows0, out_hbm.at[pl.ds(start + k_last * CHUNK, CHUNK)])

    return lookup


_lookup = _make_lookup()


def kernel(Z, W):
    W_pad = jnp.concatenate([jnp.zeros((1, EMB), jnp.float32), W], axis=0)
    return _lookup(Z, W_pad)


# speedup vs baseline: 5.6927x; 1.0050x over previous
import functools

import jax
import jax.numpy as jnp
from jax import lax
from jax.experimental import pallas as pl
from jax.experimental.pallas import tpu as pltpu
from jax.experimental.pallas import tpu_sc as plsc

N_ATOMS = 100000
EMB = 128
TABLE_ROWS = 101
CHUNK = 128

_info = plsc.get_sparse_core_info()
NC = _info.num_cores
NS = _info.num_subcores
NW = NC * NS

CHUNKS_PER_W = -(-N_ATOMS // (CHUNK * NW))
SPAN = CHUNKS_PER_W * CHUNK
PAIRS = CHUNKS_PER_W // 2


def _make_lookup():
    mesh = plsc.VectorSubcoreMesh(core_axis_name="c", subcore_axis_name="s")

    @functools.partial(
        pl.kernel,
        mesh=mesh,
        out_type=jax.ShapeDtypeStruct((N_ATOMS, EMB), jnp.float32),
        scratch_types=[
            pltpu.VMEM((SPAN,), jnp.int32),
            pltpu.VMEM((CHUNK, EMB), jnp.float32),
            pltpu.VMEM((CHUNK, EMB), jnp.float32),
            pltpu.VMEM_SHARED((TABLE_ROWS, EMB), jnp.float32),
            pltpu.SemaphoreType.DMA,
            pltpu.SemaphoreType.DMA,
            pltpu.SemaphoreType.DMA,
            pltpu.SemaphoreType.DMA,
        ],
    )
    def lookup(z_hbm, table_hbm, out_hbm, idx_all, rows0, rows1,
               table_sh, gsem0, gsem1, wsem0, wsem1):
        sid = lax.axis_index("s")
        wid = sid * NC + lax.axis_index("c")

        @pl.when(sid == 0)
        def _():
            pltpu.sync_copy(table_hbm, table_sh.at[pl.ds(1, TABLE_ROWS - 1)])

        start = jnp.minimum(wid * SPAN, N_ATOMS - SPAN)
        pltpu.sync_copy(z_hbm.at[pl.ds(start, SPAN)], idx_all)

        plsc.subcore_barrier()

        rows = (rows0, rows1)
        gsem = (gsem0, gsem1)
        wsem = (wsem0, wsem1)

        def issue_gather(k, b):
            pltpu.async_copy(
                table_sh.at[idx_all.at[pl.ds(k * CHUNK, CHUNK)]],
                rows[b], gsem[b])

        def drain(sem, b):
            pltpu.make_async_copy(out_hbm.at[pl.ds(0, CHUNK)], rows[b],
                                  sem).wait()

        issue_gather(0, 0)

        def pair_body(p, carry):
            for b in range(2):
                k = 2 * p + b
                if b == 0:
                    @pl.when(p > 0)
                    def _():
                        drain(wsem[1], 1)
                else:
                    drain(wsem[0], 0)
                issue_gather(k + 1, 1 - b)
                drain(gsem[b], b)
                pltpu.async_copy(rows[b],
                                 out_hbm.at[pl.ds(start + k * CHUNK, CHUNK)],
                                 wsem[b])
            return carry

        lax.fori_loop(0, PAIRS, pair_body, 0)

        k_last = 2 * PAIRS
        drain(wsem[1], 1)
        drain(gsem[0], 0)
        pltpu.sync_copy(rows0, out_hbm.at[pl.ds(start + k_last * CHUNK, CHUNK)])

    return lookup


_lookup = _make_lookup()


def kernel(Z, W):
    return _lookup(Z, W)
